# baseline (speedup 1.0000x reference)
import functools

import jax
import jax.numpy as jnp
from jax import lax
from jax.experimental import pallas as pl
from jax.experimental.pallas import tpu as pltpu
from jax.experimental.pallas import tpu_sc as plsc

_BS = 16
_N = 8192
_L = 100
_LANES = 16
_VREGS = _N // _LANES
_LEVELS = 9


def _sort_body(p_hbm, q_hbm, out_hbm, buf):
    wid = lax.axis_index("s") * 2 + lax.axis_index("c")

    @pl.when(wid < _BS)
    def _():
        pltpu.sync_copy(p_hbm.at[wid], buf)

    @pl.when(wid >= _BS)
    def _():
        pltpu.sync_copy(q_hbm.at[wid - _BS], buf)

    def _ld(i):
        return buf[pl.ds(i * _LANES, _LANES)]

    def _st(i, v):
        buf[pl.ds(i * _LANES, _LANES)] = v

    @plsc.parallel_loop(0, _VREGS // 2, unroll=8)
    def _(m):
        i = m << 1
        a = _ld(i)
        b = _ld(i + 1)
        sa, _ = plsc.sort_key_val(a, a)
        sd, _ = plsc.sort_key_val(b, b, descending=True)
        _st(i, sa)
        _st(i + 1, sd)

    for lvl in range(1, _LEVELS + 1):
        final = lvl >= _LEVELS
        for tlog in range(lvl - 1, 0, -1):
            t = 1 << tlog
            if final:

                @plsc.parallel_loop(0, _VREGS // 2, unroll=8)
                def _(p, tlog=tlog, t=t):
                    q = p >> tlog
                    r = p & (t - 1)
                    i = (q << (tlog + 1)) + r
                    a = _ld(i)
                    b = _ld(i + t)
                    _st(i, jnp.minimum(a, b))
                    _st(i + t, jnp.maximum(a, b))

            else:

                @plsc.parallel_loop(0, _VREGS // 4, unroll=8)
                def _(m, tlog=tlog, t=t, lvl=lvl):
                    rp = m >> (lvl - 1)
                    w = m & ((1 << (lvl - 1)) - 1)
                    q = w >> tlog
                    r = w & (t - 1)
                    ia = (rp << (lvl + 1)) + (q << (tlog + 1)) + r
                    idd = ia + (1 << lvl)
                    a = _ld(ia)
                    b = _ld(ia + t)
                    _st(ia, jnp.minimum(a, b))
                    _st(ia + t, jnp.maximum(a, b))
                    c = _ld(idd)
                    d = _ld(idd + t)
                    _st(idd, jnp.maximum(c, d))
                    _st(idd + t, jnp.minimum(c, d))

        if final:

            @plsc.parallel_loop(0, _VREGS // 2, unroll=8)
            def _(p):
                i = p << 1
                a = _ld(i)
                b = _ld(i + 1)
                slo, _ = plsc.sort_key_val(jnp.minimum(a, b), a)
                shi, _ = plsc.sort_key_val(jnp.maximum(a, b), a)
                _st(i, slo)
                _st(i + 1, shi)

        else:

            @plsc.parallel_loop(0, _VREGS // 4, unroll=4)
            def _(m, lvl=lvl):
                rp = m >> (lvl - 1)
                w = m & ((1 << (lvl - 1)) - 1)
                ia = (rp << (lvl + 1)) + (w << 1)
                idd = ia + (1 << lvl)
                a = _ld(ia)
                b = _ld(ia + 1)
                slo, _ = plsc.sort_key_val(jnp.minimum(a, b), a)
                shi, _ = plsc.sort_key_val(jnp.maximum(a, b), a)
                _st(ia, slo)
                _st(ia + 1, shi)
                c = _ld(idd)
                d = _ld(idd + 1)
                shi2, _ = plsc.sort_key_val(jnp.maximum(c, d), c, descending=True)
                slo2, _ = plsc.sort_key_val(jnp.minimum(c, d), c, descending=True)
                _st(idd, shi2)
                _st(idd + 1, slo2)

    pltpu.sync_copy(buf, out_hbm.at[wid])


_sort_call = functools.partial(
    pl.kernel,
    out_type=jax.ShapeDtypeStruct((2 * _BS, _N), jnp.float32),
    mesh=plsc.VectorSubcoreMesh(core_axis_name="c", subcore_axis_name="s"),
    scratch_types=[pltpu.VMEM((_N,), jnp.float32)],
    compiler_params=pltpu.CompilerParams(needs_layout_passes=False),
)(_sort_body)


def _reduce_body(spq_ref, th_ref, out_ref):
    d = spq_ref[0:_BS, :] - spq_ref[_BS : 2 * _BS, :]
    d2 = jnp.sum(d * d, axis=1) * jnp.float32(1.0 / _N)
    t = th_ref[...]
    tn = t / jnp.sqrt(t * t)
    f = jnp.sum(tn * tn, axis=1) * jnp.float32(1.0 / _L)
    out_ref[...] = jnp.sum(jnp.sqrt(d2 * f)).reshape(1, 1)


def kernel(P_batch, Q_batch, thetas):
    sorted_pq = _sort_call(P_batch, Q_batch)
    th = thetas.reshape(_BS, _L)
    out = pl.pallas_call(
        _reduce_body,
        out_shape=jax.ShapeDtypeStruct((1, 1), jnp.float32),
    )(sorted_pq, th)
    return out[0, 0]


# speedup vs baseline: 1.2880x; 1.2880x over previous
import functools

import jax
import jax.numpy as jnp
from jax import lax
from jax.experimental import pallas as pl
from jax.experimental.pallas import tpu as pltpu
from jax.experimental.pallas import tpu_sc as plsc

_BS = 16
_N = 8192
_L = 100
_LANES = 16
_VREGS = _N // _LANES
_LEVELS = 9


def _sort_body(p_hbm, q_hbm, out_hbm, buf):
    wid = lax.axis_index("s") * 2 + lax.axis_index("c")

    @pl.when(wid < _BS)
    def _():
        pltpu.sync_copy(p_hbm.at[wid], buf)

    @pl.when(wid >= _BS)
    def _():
        pltpu.sync_copy(q_hbm.at[wid - _BS], buf)

    def _ld(i):
        return buf[pl.ds(i * _LANES, _LANES)]

    def _st(i, v):
        buf[pl.ds(i * _LANES, _LANES)] = v

    def _srt(v, asc):
        sk, _ = plsc.sort_key_val(v, v, descending=not asc)
        return sk

    def _ce(v, k, kk, asc):
        a, b = v[k], v[kk]
        lo, hi = jnp.minimum(a, b), jnp.maximum(a, b)
        v[k], v[kk] = (lo, hi) if asc else (hi, lo)

    def _low_group(base, gpar):
        v = [_ld(base + k) for k in range(16)]
        for k in range(16):
            v[k] = _srt(v[k], k % 2 == 0)
        for lvl in (1, 2, 3, 4):
            def asc_of(k, lvl=lvl, gpar=gpar):
                return ((k >> lvl) & 1) == 0 if lvl < 4 else (gpar == 0)

            for tlog in range(lvl - 1, 0, -1):
                t = 1 << tlog
                for k in range(16):
                    if (k >> tlog) & 1 == 0:
                        _ce(v, k, k | t, asc_of(k))
            for k in range(0, 16, 2):
                asc = asc_of(k)
                _ce(v, k, k + 1, asc)
                v[k] = _srt(v[k], asc)
                v[k + 1] = _srt(v[k + 1], asc)
        for k in range(16):
            _st(base + k, v[k])

    def _chunk_ce(base, step, nbits, asc):
        n = 1 << nbits
        v = [_ld(base + j * step) for j in range(n)]
        for bb in range(nbits - 1, -1, -1):
            jt = 1 << bb
            for j in range(n):
                if (j >> bb) & 1 == 0:
                    _ce(v, j, j | jt, asc)
        for j in range(n):
            _st(base + j * step, v[j])

    def _bottom_group(base, asc):
        v = [_ld(base + k) for k in range(16)]
        for tlog in (3, 2, 1):
            t = 1 << tlog
            for k in range(16):
                if (k >> tlog) & 1 == 0:
                    _ce(v, k, k | t, asc)
        for k in range(0, 16, 2):
            _ce(v, k, k + 1, asc)
            v[k] = _srt(v[k], asc)
            v[k + 1] = _srt(v[k + 1], asc)
        for k in range(16):
            _st(base + k, v[k])

    @plsc.parallel_loop(0, _VREGS // 32, unroll=1)
    def _(u):
        _low_group((u << 5), 0)
        _low_group((u << 5) + 16, 1)

    for lvl in (5, 6, 7, 8):
        nb = lvl - 4
        rp_count = _VREGS >> (lvl + 1)

        @plsc.parallel_loop(0, rp_count * 16, unroll=max(1, 4 >> (nb - 1)))
        def _(m, lvl=lvl, nb=nb):
            rp = m >> 4
            w = m & 15
            ba = (rp << (lvl + 1)) + w
            _chunk_ce(ba, 16, nb, True)
            _chunk_ce(ba + (1 << lvl), 16, nb, False)

        @plsc.parallel_loop(0, rp_count << nb, unroll=1)
        def _(m, lvl=lvl, nb=nb):
            rp = m >> nb
            w = m & ((1 << nb) - 1)
            ba = (rp << (lvl + 1)) + (w << 4)
            _bottom_group(ba, True)
            _bottom_group(ba + (1 << lvl), False)

    @plsc.parallel_loop(0, 32, unroll=1)
    def _(m):
        _chunk_ce(m, 32, 4, True)

    @plsc.parallel_loop(0, 16, unroll=1)
    def _(m):
        base = m << 5
        for k in range(16):
            a = _ld(base + k)
            b = _ld(base + k + 16)
            _st(base + k, jnp.minimum(a, b))
            _st(base + k + 16, jnp.maximum(a, b))

    @plsc.parallel_loop(0, 32, unroll=1)
    def _(m):
        _bottom_group(m << 4, True)

    pltpu.sync_copy(buf, out_hbm.at[wid])


_sort_call = functools.partial(
    pl.kernel,
    out_type=jax.ShapeDtypeStruct((2 * _BS, _N), jnp.float32),
    mesh=plsc.VectorSubcoreMesh(core_axis_name="c", subcore_axis_name="s"),
    scratch_types=[pltpu.VMEM((_N,), jnp.float32)],
    compiler_params=pltpu.CompilerParams(needs_layout_passes=False),
)(_sort_body)


def _reduce_body(spq_ref, th_ref, out_ref):
    d = spq_ref[0:_BS, :] - spq_ref[_BS : 2 * _BS, :]
    d2 = jnp.sum(d * d, axis=1) * jnp.float32(1.0 / _N)
    t = th_ref[...]
    tn = t / jnp.sqrt(t * t)
    f = jnp.sum(tn * tn, axis=1) * jnp.float32(1.0 / _L)
    out_ref[...] = jnp.sum(jnp.sqrt(d2 * f)).reshape(1, 1)


def kernel(P_batch, Q_batch, thetas):
    sorted_pq = _sort_call(P_batch, Q_batch)
    th = thetas.reshape(_BS, _L)
    out = pl.pallas_call(
        _reduce_body,
        out_shape=jax.ShapeDtypeStruct((1, 1), jnp.float32),
    )(sorted_pq, th)
    return out[0, 0]


# speedup vs baseline: 1.2907x; 1.0020x over previous
import functools

import jax
import jax.numpy as jnp
from jax import lax
from jax.experimental import pallas as pl
from jax.experimental.pallas import tpu as pltpu
from jax.experimental.pallas import tpu_sc as plsc

_BS = 16
_N = 8192
_L = 100
_LANES = 16
_VREGS = _N // _LANES
_LEVELS = 9


def _sort_body(p_hbm, q_hbm, out_hbm, buf):
    wid = lax.axis_index("s") * 2 + lax.axis_index("c")

    @pl.when(wid < _BS)
    def _():
        pltpu.sync_copy(p_hbm.at[wid], buf)

    @pl.when(wid >= _BS)
    def _():
        pltpu.sync_copy(q_hbm.at[wid - _BS], buf)

    def _ld(i):
        return buf[pl.ds(i * _LANES, _LANES)]

    def _st(i, v):
        buf[pl.ds(i * _LANES, _LANES)] = v

    def _srt(v, asc):
        sk, _ = plsc.sort_key_val(v, v, descending=not asc)
        return sk

    def _ce(v, k, kk, asc):
        a, b = v[k], v[kk]
        lo, hi = jnp.minimum(a, b), jnp.maximum(a, b)
        v[k], v[kk] = (lo, hi) if asc else (hi, lo)

    def _low_group(base, gpar):
        v = [_ld(base + k) for k in range(16)]
        for k in range(16):
            v[k] = _srt(v[k], k % 2 == 0)
        for lvl in (1, 2, 3, 4):
            def asc_of(k, lvl=lvl, gpar=gpar):
                return ((k >> lvl) & 1) == 0 if lvl < 4 else (gpar == 0)

            for tlog in range(lvl - 1, 0, -1):
                t = 1 << tlog
                for k in range(16):
                    if (k >> tlog) & 1 == 0:
                        _ce(v, k, k | t, asc_of(k))
            for k in range(0, 16, 2):
                asc = asc_of(k)
                _ce(v, k, k + 1, asc)
                v[k] = _srt(v[k], asc)
                v[k + 1] = _srt(v[k + 1], asc)
        for k in range(16):
            _st(base + k, v[k])

    def _chunk_ce(base, step, nbits, asc):
        n = 1 << nbits
        v = [_ld(base + j * step) for j in range(n)]
        for bb in range(nbits - 1, -1, -1):
            jt = 1 << bb
            for j in range(n):
                if (j >> bb) & 1 == 0:
                    _ce(v, j, j | jt, asc)
        for j in range(n):
            _st(base + j * step, v[j])

    def _bottom_group(base, asc):
        v = [_ld(base + k) for k in range(16)]
        for tlog in (3, 2, 1):
            t = 1 << tlog
            for k in range(16):
                if (k >> tlog) & 1 == 0:
                    _ce(v, k, k | t, asc)
        for k in range(0, 16, 2):
            _ce(v, k, k + 1, asc)
            v[k] = _srt(v[k], asc)
            v[k + 1] = _srt(v[k + 1], asc)
        for k in range(16):
            _st(base + k, v[k])

    @plsc.parallel_loop(0, _VREGS // 32, unroll=1)
    def _(u):
        _low_group((u << 5), 0)
        _low_group((u << 5) + 16, 1)

    for lvl in (5, 6, 7, 8):
        nb = lvl - 4
        rp_count = _VREGS >> (lvl + 1)

        @plsc.parallel_loop(0, rp_count * 16, unroll=max(1, 4 >> (nb - 1)))
        def _(m, lvl=lvl, nb=nb):
            rp = m >> 4
            w = m & 15
            ba = (rp << (lvl + 1)) + w
            _chunk_ce(ba, 16, nb, True)
            _chunk_ce(ba + (1 << lvl), 16, nb, False)

        @plsc.parallel_loop(0, rp_count << nb, unroll=1)
        def _(m, lvl=lvl, nb=nb):
            rp = m >> nb
            w = m & ((1 << nb) - 1)
            ba = (rp << (lvl + 1)) + (w << 4)
            _bottom_group(ba, True)
            _bottom_group(ba + (1 << lvl), False)

    @plsc.parallel_loop(0, 32, unroll=1)
    def _(m):
        _chunk_ce(m, 32, 4, True)

    @plsc.parallel_loop(0, 16, unroll=1)
    def _(m):
        base = m << 5
        for k in range(16):
            a = _ld(base + k)
            b = _ld(base + k + 16)
            _st(base + k, jnp.minimum(a, b))
            _st(base + k + 16, jnp.maximum(a, b))

    @plsc.parallel_loop(0, 32, unroll=1)
    def _(m):
        _bottom_group(m << 4, True)

    pltpu.sync_copy(buf, out_hbm.at[wid])


_sort_call = functools.partial(
    pl.kernel,
    out_type=jax.ShapeDtypeStruct((2 * _BS, _N), jnp.float32),
    mesh=plsc.VectorSubcoreMesh(core_axis_name="c", subcore_axis_name="s"),
    scratch_types=[pltpu.VMEM((_N,), jnp.float32)],
    compiler_params=pltpu.CompilerParams(
        needs_layout_passes=False, skip_device_barrier=True
    ),
)(_sort_body)


def _reduce_body(spq_ref, th_ref, out_ref):
    d = spq_ref[0:_BS, :] - spq_ref[_BS : 2 * _BS, :]
    d2 = jnp.sum(d * d, axis=1) * jnp.float32(1.0 / _N)
    t = th_ref[...]
    tn = t / jnp.sqrt(t * t)
    f = jnp.sum(tn * tn, axis=1) * jnp.float32(1.0 / _L)
    out_ref[...] = jnp.sum(jnp.sqrt(d2 * f)).reshape(1, 1)


def kernel(P_batch, Q_batch, thetas):
    sorted_pq = _sort_call(P_batch, Q_batch)
    th = thetas.reshape(_BS, _L)
    out = pl.pallas_call(
        _reduce_body,
        out_shape=jax.ShapeDtypeStruct((1, 1), jnp.float32),
    )(sorted_pq, th)
    return out[0, 0]
